# baseline (speedup 1.0000x reference)
import functools

import jax
import jax.numpy as jnp
from jax import lax
from jax.experimental import pallas as pl
from jax.experimental.pallas import tpu as pltpu
from jax.experimental.pallas import tpu_sc as plsc

_TILE_V = 6144
_NA = 0
_NSEM = 8


def _make_sc_gather(B, L, V, H):
    info = plsc.get_sparse_core_info()
    NC, NS = info.num_cores, info.num_subcores
    NW = NC * NS
    b_per_w = B * L // NW
    mesh = plsc.VectorSubcoreMesh(core_axis_name="c", subcore_axis_name="s")
    two_d = B == NW and b_per_w == L

    @functools.partial(
        pl.kernel,
        mesh=mesh,
        out_type=jax.ShapeDtypeStruct((B * L, H), jnp.float32),
        scratch_types=[
            pltpu.VMEM((b_per_w,), jnp.int32),
            pltpu.VMEM((b_per_w, H), jnp.float32),
            pltpu.SemaphoreType.DMA,
        ],
    )
    def gather_k(idx_hbm, table_hbm, out_hbm, idx_v, rows_v, sem):
        wid = lax.axis_index("s") * NC + lax.axis_index("c")
        base = wid * b_per_w
        if two_d:
            pltpu.sync_copy(idx_hbm.at[wid], idx_v)
        else:
            pltpu.sync_copy(idx_hbm.at[pl.ds(base, b_per_w)], idx_v)
        pltpu.async_copy(table_hbm.at[idx_v], rows_v, sem).wait()
        pltpu.sync_copy(rows_v, out_hbm.at[pl.ds(base, b_per_w)])

    return gather_k, two_d


def _head_a(ids2d, emb, w, b, tile_v, na):
    B, L = ids2d.shape
    Bt = B * L
    V, H = w.shape

    def body(ids_ref, emb_hbm, w_ref, b_ref, o_ref, x_vmem, sems):
        @pl.when(pl.program_id(0) == 0)
        def _gather():
            for i in range(Bt):
                r = ids_ref[i // L, i % L]
                pltpu.make_async_copy(
                    emb_hbm.at[pl.ds(r, 1), :],
                    x_vmem.at[pl.ds(i, 1), :],
                    sems.at[i % _NSEM],
                ).start()
            per = Bt // _NSEM
            for k in range(_NSEM):
                pltpu.make_async_copy(
                    emb_hbm.at[pl.ds(0, per), :],
                    x_vmem.at[pl.ds(k * per, per), :],
                    sems.at[k],
                ).wait()

        o_ref[...] = (
            lax.dot_general(
                x_vmem[...],
                w_ref[...],
                (((1,), (1,)), ((), ())),
                preferred_element_type=jnp.float32,
            )
            + b_ref[...][None, :]
        )

    return pl.pallas_call(
        body,
        grid=(na,),
        in_specs=[
            pl.BlockSpec(memory_space=pltpu.MemorySpace.SMEM),
            pl.BlockSpec(memory_space=pltpu.MemorySpace.HBM),
            pl.BlockSpec((tile_v, H), lambda i: (i, 0)),
            pl.BlockSpec((tile_v,), lambda i: (i,)),
        ],
        out_specs=pl.BlockSpec((Bt, tile_v), lambda i: (0, i)),
        out_shape=jax.ShapeDtypeStruct((Bt, V), jnp.float32),
        scratch_shapes=[
            pltpu.VMEM((Bt, H), jnp.float32),
            pltpu.SemaphoreType.DMA((_NSEM,)),
        ],
    )(ids2d, emb, w, b)


def _head_b(x, w, b, partial_out, tile_v, na):
    Bt, H = x.shape
    V = w.shape[0]
    nv = pl.cdiv(V, tile_v)

    def body(x_ref, w_ref, b_ref, _alias_ref, o_ref):
        o_ref[...] = (
            lax.dot_general(
                x_ref[...],
                w_ref[...],
                (((1,), (1,)), ((), ())),
                preferred_element_type=jnp.float32,
            )
            + b_ref[...][None, :]
        )

    return pl.pallas_call(
        body,
        grid=(nv - na,),
        in_specs=[
            pl.BlockSpec((Bt, H), lambda i: (0, 0)),
            pl.BlockSpec((tile_v, H), lambda i: (i + na, 0)),
            pl.BlockSpec((tile_v,), lambda i: (i + na,)),
            pl.BlockSpec(memory_space=pltpu.MemorySpace.HBM),
        ],
        out_specs=pl.BlockSpec((Bt, tile_v), lambda i: (0, i + na)),
        out_shape=jax.ShapeDtypeStruct((Bt, V), jnp.float32),
        input_output_aliases={3: 0},
    )(x, w, b, partial_out)


def _head_b_solo(x, w, b, tile_v):
    Bt, H = x.shape
    V = w.shape[0]
    nv = pl.cdiv(V, tile_v)

    def body(x_ref, w_ref, b_ref, o_ref):
        o_ref[...] = (
            lax.dot_general(
                x_ref[...],
                w_ref[...],
                (((1,), (1,)), ((), ())),
                preferred_element_type=jnp.float32,
            )
            + b_ref[...][None, :]
        )

    return pl.pallas_call(
        body,
        grid=(nv,),
        in_specs=[
            pl.BlockSpec((Bt, H), lambda i: (0, 0)),
            pl.BlockSpec((tile_v, H), lambda i: (i, 0)),
            pl.BlockSpec((tile_v,), lambda i: (i,)),
        ],
        out_specs=pl.BlockSpec((Bt, tile_v), lambda i: (0, i)),
        out_shape=jax.ShapeDtypeStruct((Bt, V), jnp.float32),
    )(x, w, b)


def kernel(input_ids, embedding, lm_head_w, lm_head_b):
    B, L = input_ids.shape
    V, H = embedding.shape
    ids2d = input_ids.astype(jnp.int32)
    gather_k, two_d = _make_sc_gather(B, L, V, H)
    embeds = gather_k(ids2d if two_d else ids2d.reshape(B * L), embedding)
    if _NA > 0:
        part = _head_a(ids2d, embedding, lm_head_w, lm_head_b, _TILE_V, _NA)
        logits = _head_b(embeds, lm_head_w, lm_head_b, part, _TILE_V, _NA)
    else:
        logits = _head_b_solo(embeds, lm_head_w, lm_head_b, _TILE_V)
    return logits.reshape(B, L, V)


# speedup vs baseline: 1.0073x; 1.0073x over previous
import functools

import jax
import jax.numpy as jnp
from jax import lax
from jax.experimental import pallas as pl
from jax.experimental.pallas import tpu as pltpu
from jax.experimental.pallas import tpu_sc as plsc

_TILE_V = 5120
_NA = 0
_NSEM = 8


def _make_sc_gather(B, L, V, H):
    info = plsc.get_sparse_core_info()
    NC, NS = info.num_cores, info.num_subcores
    NW = NC * NS
    b_per_w = B * L // NW
    mesh = plsc.VectorSubcoreMesh(core_axis_name="c", subcore_axis_name="s")
    two_d = B == NW and b_per_w == L

    @functools.partial(
        pl.kernel,
        mesh=mesh,
        out_type=jax.ShapeDtypeStruct((B * L, H), jnp.float32),
        scratch_types=[
            pltpu.VMEM((b_per_w,), jnp.int32),
            pltpu.VMEM((b_per_w, H), jnp.float32),
            pltpu.SemaphoreType.DMA,
        ],
    )
    def gather_k(idx_hbm, table_hbm, out_hbm, idx_v, rows_v, sem):
        wid = lax.axis_index("s") * NC + lax.axis_index("c")
        base = wid * b_per_w
        if two_d:
            pltpu.sync_copy(idx_hbm.at[wid], idx_v)
        else:
            pltpu.sync_copy(idx_hbm.at[pl.ds(base, b_per_w)], idx_v)
        pltpu.async_copy(table_hbm.at[idx_v], rows_v, sem).wait()
        pltpu.sync_copy(rows_v, out_hbm.at[pl.ds(base, b_per_w)])

    return gather_k, two_d


def _head_a(ids2d, emb, w, b, tile_v, na):
    B, L = ids2d.shape
    Bt = B * L
    V, H = w.shape

    def body(ids_ref, emb_hbm, w_ref, b_ref, o_ref, x_vmem, sems):
        @pl.when(pl.program_id(0) == 0)
        def _gather():
            for i in range(Bt):
                r = ids_ref[i // L, i % L]
                pltpu.make_async_copy(
                    emb_hbm.at[pl.ds(r, 1), :],
                    x_vmem.at[pl.ds(i, 1), :],
                    sems.at[i % _NSEM],
                ).start()
            per = Bt // _NSEM
            for k in range(_NSEM):
                pltpu.make_async_copy(
                    emb_hbm.at[pl.ds(0, per), :],
                    x_vmem.at[pl.ds(k * per, per), :],
                    sems.at[k],
                ).wait()

        o_ref[...] = (
            lax.dot_general(
                x_vmem[...],
                w_ref[...],
                (((1,), (1,)), ((), ())),
                preferred_element_type=jnp.float32,
            )
            + b_ref[...][None, :]
        )

    return pl.pallas_call(
        body,
        grid=(na,),
        in_specs=[
            pl.BlockSpec(memory_space=pltpu.MemorySpace.SMEM),
            pl.BlockSpec(memory_space=pltpu.MemorySpace.HBM),
            pl.BlockSpec((tile_v, H), lambda i: (i, 0)),
            pl.BlockSpec((tile_v,), lambda i: (i,)),
        ],
        out_specs=pl.BlockSpec((Bt, tile_v), lambda i: (0, i)),
        out_shape=jax.ShapeDtypeStruct((Bt, V), jnp.float32),
        scratch_shapes=[
            pltpu.VMEM((Bt, H), jnp.float32),
            pltpu.SemaphoreType.DMA((_NSEM,)),
        ],
    )(ids2d, emb, w, b)


def _head_b(x, w, b, partial_out, tile_v, na):
    Bt, H = x.shape
    V = w.shape[0]
    nv = pl.cdiv(V, tile_v)

    def body(x_ref, w_ref, b_ref, _alias_ref, o_ref):
        o_ref[...] = (
            lax.dot_general(
                x_ref[...],
                w_ref[...],
                (((1,), (1,)), ((), ())),
                preferred_element_type=jnp.float32,
            )
            + b_ref[...][None, :]
        )

    return pl.pallas_call(
        body,
        grid=(nv - na,),
        in_specs=[
            pl.BlockSpec((Bt, H), lambda i: (0, 0)),
            pl.BlockSpec((tile_v, H), lambda i: (i + na, 0)),
            pl.BlockSpec((tile_v,), lambda i: (i + na,)),
            pl.BlockSpec(memory_space=pltpu.MemorySpace.HBM),
        ],
        out_specs=pl.BlockSpec((Bt, tile_v), lambda i: (0, i + na)),
        out_shape=jax.ShapeDtypeStruct((Bt, V), jnp.float32),
        input_output_aliases={3: 0},
    )(x, w, b, partial_out)


def _head_b_solo(x, w, b, tile_v):
    Bt, H = x.shape
    V = w.shape[0]
    nv = pl.cdiv(V, tile_v)

    def body(x_ref, w_ref, b_ref, o_ref):
        o_ref[...] = (
            lax.dot_general(
                x_ref[...],
                w_ref[...],
                (((1,), (1,)), ((), ())),
                preferred_element_type=jnp.float32,
            )
            + b_ref[...][None, :]
        )

    return pl.pallas_call(
        body,
        grid=(nv,),
        in_specs=[
            pl.BlockSpec((Bt, H), lambda i: (0, 0)),
            pl.BlockSpec((tile_v, H), lambda i: (i, 0)),
            pl.BlockSpec((tile_v,), lambda i: (i,)),
        ],
        out_specs=pl.BlockSpec((Bt, tile_v), lambda i: (0, i)),
        out_shape=jax.ShapeDtypeStruct((Bt, V), jnp.float32),
    )(x, w, b)


def kernel(input_ids, embedding, lm_head_w, lm_head_b):
    B, L = input_ids.shape
    V, H = embedding.shape
    ids2d = input_ids if input_ids.dtype == jnp.int32 else input_ids.astype(jnp.int32)
    gather_k, two_d = _make_sc_gather(B, L, V, H)
    embeds = gather_k(ids2d if two_d else ids2d.reshape(B * L), embedding)
    if _NA > 0:
        part = _head_a(ids2d, embedding, lm_head_w, lm_head_b, _TILE_V, _NA)
        logits = _head_b(embeds, lm_head_w, lm_head_b, part, _TILE_V, _NA)
    else:
        logits = _head_b_solo(embeds, lm_head_w, lm_head_b, _TILE_V)
    return logits.reshape(B, L, V)
